# baseline (speedup 1.0000x reference)
import functools

import jax
import jax.numpy as jnp
from jax import lax
from jax.experimental import pallas as pl
from jax.experimental.pallas import tpu as pltpu

V, H, E = 32, 1024, 1536
N = 64
ROW = H * E
CH = 4
D = ROW // CH


def _tc_body(ids_smem, table_any, out_any, sem):
    def start(i, carry):
        r = ids_smem[i // CH]
        c = i % CH
        pltpu.make_async_copy(
            table_any.at[r, pl.ds(c * D, D)],
            out_any.at[i // CH, pl.ds(c * D, D)],
            sem,
        ).start()
        return carry

    lax.fori_loop(0, N * CH, start, 0)

    def drain(i, carry):
        pltpu.make_async_copy(
            table_any.at[0, pl.ds(0, D)],
            out_any.at[0, pl.ds(0, D)],
            sem,
        ).wait()
        return carry

    lax.fori_loop(0, N * CH, drain, 0)


@jax.jit
def _tc_gather(ids, table):
    return pl.pallas_call(
        _tc_body,
        grid_spec=pltpu.PrefetchScalarGridSpec(
            num_scalar_prefetch=1,
            grid=(1,),
            in_specs=[pl.BlockSpec(memory_space=pl.ANY)],
            out_specs=pl.BlockSpec(memory_space=pl.ANY),
            scratch_shapes=[pltpu.SemaphoreType.DMA],
        ),
        out_shape=jax.ShapeDtypeStruct((N, ROW), jnp.float32),
    )(ids, table)


def kernel(cat_ids, W):
    table = W.reshape(V, ROW)
    out2 = _tc_gather(cat_ids.astype(jnp.int32), table)
    return out2.reshape(N, H, E)


# speedup vs baseline: 49.1864x; 49.1864x over previous
import functools

import jax
import jax.numpy as jnp
from jax.experimental import pallas as pl
from jax.experimental.pallas import tpu as pltpu

V, H, E = 32, 1024, 1536
N = 64
BH = 512
NB = H // BH


def _copy_body(ids_smem, in_ref, out_ref):
    out_ref[...] = in_ref[...]


@jax.jit
def _tc_gather(ids, table):
    return pl.pallas_call(
        _copy_body,
        grid_spec=pltpu.PrefetchScalarGridSpec(
            num_scalar_prefetch=1,
            grid=(N, NB),
            in_specs=[
                pl.BlockSpec((1, BH, E), lambda i, j, ids: (ids[i], j, 0)),
            ],
            out_specs=pl.BlockSpec((1, BH, E), lambda i, j, ids: (i, j, 0)),
        ),
        out_shape=jax.ShapeDtypeStruct((N, H, E), jnp.float32),
    )(ids, table)


def kernel(cat_ids, W):
    return _tc_gather(cat_ids.astype(jnp.int32), W)


# speedup vs baseline: 50.8105x; 1.0330x over previous
import functools

import jax
import jax.numpy as jnp
from jax.experimental import pallas as pl
from jax.experimental.pallas import tpu as pltpu

V, H, E = 32, 1024, 1536
N = 64
BH = 1024
NB = H // BH


def _copy_body(ids_smem, in_ref, out_ref):
    out_ref[...] = in_ref[...]


@jax.jit
def _tc_gather(ids, table):
    return pl.pallas_call(
        _copy_body,
        grid_spec=pltpu.PrefetchScalarGridSpec(
            num_scalar_prefetch=1,
            grid=(N, NB),
            in_specs=[
                pl.BlockSpec((1, BH, E), lambda i, j, ids: (ids[i], j, 0)),
            ],
            out_specs=pl.BlockSpec((1, BH, E), lambda i, j, ids: (i, j, 0)),
        ),
        out_shape=jax.ShapeDtypeStruct((N, H, E), jnp.float32),
    )(ids, table)


def kernel(cat_ids, W):
    return _tc_gather(cat_ids.astype(jnp.int32), W)
